# Initial kernel scaffold; baseline (speedup 1.0000x reference)
import jax
import jax.numpy as jnp
from jax.experimental import pallas as pl


def kernel(x, h0, c0, wih1_t, wcomb, bias, fc_w, fc_b):
    raise NotImplementedError("write your pallas kernel here")


# speedup vs baseline: 1.0541x; 1.0541x over previous
import functools

import jax
import jax.numpy as jnp
from jax.experimental import pallas as pl
from jax.experimental.pallas import tpu as pltpu


def _lstm_fc_kernel(x_ref, h0_ref, c0_ref, wih1_ref, wcomb_ref, bias_ref,
                    fcw_ref, fcb_ref,
                    out_ref, hN_ref, cN_ref,
                    pre_scr, seq_scr,
                    *, T, BB):
    L, _, H = h0_ref.shape
    G = 4 * H
    W = L * G

    lane = jax.lax.broadcasted_iota(jnp.int32, (1, G), 1)
    is_g = (lane // H) == 2
    a_scale = jnp.where(is_g, 1.0, 0.5).astype(jnp.float32)
    a_add = jnp.where(is_g, 0.0, 0.5).astype(jnp.float32)

    x2d = x_ref[...].reshape(T * BB, x_ref.shape[-1])
    pre_scr[...] = (jnp.dot(x2d, wih1_ref[...],
                            preferred_element_type=jnp.float32)
                    + bias_ref[:, 0:G])

    wcomb = wcomb_ref[...]
    bias = bias_ref[...]

    h_st = [h0_ref[l] for l in range(L)]
    c_st = [c0_ref[l] for l in range(L)]

    for r in range(T + L - 1):
        z = jnp.dot(jnp.concatenate(h_st, axis=1), wcomb,
                    preferred_element_type=jnp.float32)
        for l in range(L):
            t = r - l
            if 0 <= t < T:
                zl = z[:, l * G:(l + 1) * G]
                if l == 0:
                    zl = zl + pre_scr[pl.ds(t * BB, BB), :]
                else:
                    zl = zl + bias[:, l * G:(l + 1) * G]
                y = jnp.tanh(zl * a_scale)
                a = y * a_scale + a_add
                i_g = a[:, 0 * H:1 * H]
                f_g = a[:, 1 * H:2 * H]
                g_g = a[:, 2 * H:3 * H]
                o_g = a[:, 3 * H:4 * H]
                c_new = f_g * c_st[l] + i_g * g_g
                h_new = o_g * jnp.tanh(c_new)
                c_st[l] = c_new
                h_st[l] = h_new
                if l == L - 1:
                    seq_scr[pl.ds(t * BB, BB), :] = h_new

    for l in range(L):
        hN_ref[l] = h_st[l]
        cN_ref[l] = c_st[l]

    o = jnp.dot(seq_scr[...], fcw_ref[...],
                preferred_element_type=jnp.float32) + fcb_ref[...]
    out_ref[...] = o.reshape(T, BB, fcw_ref.shape[-1])


@jax.jit
def kernel(x, h0, c0, wih1_t, wcomb, bias, fc_w, fc_b):
    B, T, I = x.shape
    L, _, H = h0.shape
    O_pad = fc_w.shape[-1]
    O = 128

    NB = 2
    BB = B // NB

    x_tm = jnp.transpose(x, (1, 0, 2))

    kern = functools.partial(_lstm_fc_kernel, T=T, BB=BB)

    out_tm, hN, cN = pl.pallas_call(
        kern,
        out_shape=(jax.ShapeDtypeStruct((T, B, O_pad), jnp.float32),
                   jax.ShapeDtypeStruct((L, B, H), jnp.float32),
                   jax.ShapeDtypeStruct((L, B, H), jnp.float32)),
        grid=(NB,),
        in_specs=[
            pl.BlockSpec((T, BB, I), lambda j: (0, j, 0)),
            pl.BlockSpec((L, BB, H), lambda j: (0, j, 0)),
            pl.BlockSpec((L, BB, H), lambda j: (0, j, 0)),
            pl.BlockSpec(wih1_t.shape, lambda j: (0, 0)),
            pl.BlockSpec(wcomb.shape, lambda j: (0, 0)),
            pl.BlockSpec(bias.shape, lambda j: (0, 0)),
            pl.BlockSpec(fc_w.shape, lambda j: (0, 0)),
            pl.BlockSpec(fc_b.shape, lambda j: (0, 0)),
        ],
        out_specs=[
            pl.BlockSpec((T, BB, O_pad), lambda j: (0, j, 0)),
            pl.BlockSpec((L, BB, H), lambda j: (0, j, 0)),
            pl.BlockSpec((L, BB, H), lambda j: (0, j, 0)),
        ],
        scratch_shapes=[
            pltpu.VMEM((T * BB, 4 * H), jnp.float32),
            pltpu.VMEM((T * BB, H), jnp.float32),
        ],
        compiler_params=pltpu.CompilerParams(
            dimension_semantics=("parallel",)),
    )(x_tm, h0, c0, wih1_t, wcomb, bias, fc_w, fc_b)

    out = jnp.transpose(out_tm[:, :, :O], (1, 0, 2)).reshape(B * T, O)
    return out, (hN, cN)
